# baseline (speedup 1.0000x reference)
import functools

import jax
import jax.numpy as jnp
from jax import lax
from jax.experimental import pallas as pl
from jax.experimental.pallas import tpu as pltpu
from jax.experimental.pallas import tpu_sc as plsc

N_NODES = 10000
N_EDGES = 320000
FEATURE = 128
HIDDEN = 128
CLASSES = 64

NC = 2
NS = 16
NW = NC * NS
WIN = 128
WPW = 80
E_PAD = NW * WPW * WIN
ACC_ROWS = N_NODES + 16
STRIPE = ACC_ROWS // NS

ROW_BLK = 1000


def _make_sc_partials(width, n_tab):
  mesh = plsc.VectorSubcoreMesh(core_axis_name="c", subcore_axis_name="s")
  wout = N_NODES // NS

  @functools.partial(
      pl.kernel,
      mesh=mesh,
      compiler_params=pltpu.CompilerParams(use_tc_tiling_on_sc=False),
      out_type=jax.ShapeDtypeStruct((NC, n_tab, N_NODES, width), jnp.float32),
      scratch_types=[
          pltpu.VMEM((WPW, WIN), jnp.int32),
          pltpu.VMEM((2, WIN), jnp.int32),
          pltpu.VMEM((2, WIN), jnp.int32),
          pltpu.VMEM((WIN, width), jnp.float32),
          pltpu.VMEM((WIN, width), jnp.float32),
          pltpu.VMEM_SHARED((N_NODES, width), jnp.float32),
          pltpu.VMEM_SHARED((ACC_ROWS, width), jnp.float32),
          pltpu.SemaphoreType.DMA,
          pltpu.SemaphoreType.DMA,
      ],
  )
  def sc_kernel(*refs):
    y_hbms = refs[:n_tab]
    pk_hbm, out_hbm, pk_v, src_i, dst_i, buf, buf_b, y_s, acc, sem, sem_b = (
        refs[n_tab:])
    c = lax.axis_index("c")
    s = lax.axis_index("s")
    wid = c * NS + s

    pltpu.sync_copy(pk_hbm.at[wid], pk_v)

    mask16 = jnp.full((16,), 0xFFFF, jnp.int32)
    sh16 = jnp.full((16,), 16, jnp.int32)

    def _unpack(j, slot):
      for cc in range(WIN // 16):
        sl = pl.ds(cc * 16, 16)
        v = pk_v[j, sl]
        src_i[slot, sl] = v & mask16
        dst_i[slot, sl] = lax.shift_right_logical(v, sh16)

    zero16 = jnp.zeros((16,), jnp.float32)
    load = N_NODES // NS

    def _pass(y_hbm, tab):
      pltpu.sync_copy(y_hbm.at[pl.ds(s * load, load)],
                      y_s.at[pl.ds(s * load, load)])

      @pl.loop(0, WIN)
      def _(r):
        for cc in range(width // 16):
          buf[r, pl.ds(cc * 16, 16)] = zero16

      base = s * STRIPE
      for k in range(STRIPE // WIN):
        pltpu.sync_copy(buf, acc.at[pl.ds(base + k * WIN, WIN)])
      rem = STRIPE % WIN
      if rem:
        pltpu.sync_copy(buf.at[pl.ds(0, rem)],
                        acc.at[pl.ds(base + (STRIPE // WIN) * WIN, rem)])

      plsc.subcore_barrier()

      def _fire(slot, b, sm):
        pltpu.make_async_copy(y_s.at[src_i.at[slot]], b, sm).start()

      def _drain(slot, b, sm):
        pltpu.make_async_copy(y_s.at[src_i.at[slot]], b, sm).wait()

      _unpack(0, 0)
      _fire(0, buf, sem)

      @pl.loop(0, WPW // 2)
      def _(p):
        j = 2 * p
        _unpack(j + 1, 1)
        _fire(1, buf_b, sem_b)
        _drain(0, buf, sem)
        pltpu.sync_copy(buf, acc.at[dst_i.at[0]], add=True)

        @pl.when(j + 2 < WPW)
        def _():
          _unpack(j + 2, 0)
          _fire(0, buf, sem)

        _drain(1, buf_b, sem_b)
        pltpu.sync_copy(buf_b, acc.at[dst_i.at[1]], add=True)

      plsc.subcore_barrier()

      pltpu.sync_copy(acc.at[pl.ds(s * wout, wout)],
                      out_hbm.at[c, tab, pl.ds(s * wout, wout)])
      plsc.subcore_barrier()

    for tab in range(n_tab):
      _pass(y_hbms[tab], tab)

  return sc_kernel


W64 = 64
_sc_partials_l1 = _make_sc_partials(W64, 2)
_sc_partials_l2 = _make_sc_partials(W64, 1)


def _dotT(a, w):
  return lax.dot_general(a, w, (((1,), (1,)), ((), ())),
                         preferred_element_type=jnp.float32)


def _tc_pre_body(x_ref, wra_ref, wrb_ref, wroot_ref, b_ref,
                 ya_ref, yb_ref, xr_ref):
  xb = x_ref[...]
  ya_ref[...] = _dotT(xb, wra_ref[...])
  yb_ref[...] = _dotT(xb, wrb_ref[...])
  xr_ref[...] = _dotT(xb, wroot_ref[...]) + b_ref[...]


_tc_pre = pl.pallas_call(
    _tc_pre_body,
    grid=(N_NODES // ROW_BLK,),
    in_specs=[
        pl.BlockSpec((ROW_BLK, FEATURE), lambda i: (i, 0)),
        pl.BlockSpec((W64, FEATURE), lambda i: (0, 0)),
        pl.BlockSpec((W64, FEATURE), lambda i: (0, 0)),
        pl.BlockSpec((HIDDEN, FEATURE), lambda i: (0, 0)),
        pl.BlockSpec((1, HIDDEN), lambda i: (0, 0)),
    ],
    out_specs=[
        pl.BlockSpec((ROW_BLK, W64), lambda i: (i, 0)),
        pl.BlockSpec((ROW_BLK, W64), lambda i: (i, 0)),
        pl.BlockSpec((ROW_BLK, HIDDEN), lambda i: (i, 0)),
    ],
    out_shape=[
        jax.ShapeDtypeStruct((N_NODES, W64), jnp.float32),
        jax.ShapeDtypeStruct((N_NODES, W64), jnp.float32),
        jax.ShapeDtypeStruct((N_NODES, HIDDEN), jnp.float32),
    ],
)


def _tc_mid_body(pa0_ref, pa1_ref, pb0_ref, pb1_ref, xr_ref,
                 wrel_ref, wroot_ref, b_ref, y2_ref, hr_ref):
  ha = pa0_ref[...] + pa1_ref[...] + xr_ref[:, :W64]
  hb = pb0_ref[...] + pb1_ref[...] + xr_ref[:, W64:]
  h = jnp.maximum(jnp.concatenate([ha, hb], axis=1), 0.0)
  y2_ref[...] = _dotT(h, wrel_ref[...])
  hr_ref[...] = _dotT(h, wroot_ref[...]) + b_ref[...]


_tc_mid = pl.pallas_call(
    _tc_mid_body,
    grid=(N_NODES // ROW_BLK,),
    in_specs=[
        pl.BlockSpec((ROW_BLK, W64), lambda i: (i, 0)),
        pl.BlockSpec((ROW_BLK, W64), lambda i: (i, 0)),
        pl.BlockSpec((ROW_BLK, W64), lambda i: (i, 0)),
        pl.BlockSpec((ROW_BLK, W64), lambda i: (i, 0)),
        pl.BlockSpec((ROW_BLK, HIDDEN), lambda i: (i, 0)),
        pl.BlockSpec((CLASSES, HIDDEN), lambda i: (0, 0)),
        pl.BlockSpec((CLASSES, HIDDEN), lambda i: (0, 0)),
        pl.BlockSpec((1, CLASSES), lambda i: (0, 0)),
    ],
    out_specs=[
        pl.BlockSpec((ROW_BLK, CLASSES), lambda i: (i, 0)),
        pl.BlockSpec((ROW_BLK, CLASSES), lambda i: (i, 0)),
    ],
    out_shape=[
        jax.ShapeDtypeStruct((N_NODES, CLASSES), jnp.float32),
        jax.ShapeDtypeStruct((N_NODES, CLASSES), jnp.float32),
    ],
)


def _tc_post_body(p0_ref, p1_ref, hr_ref, o_ref):
  z = p0_ref[...] + p1_ref[...] + hr_ref[...]
  m = jnp.max(z, axis=1, keepdims=True)
  zs = z - m
  lse = jnp.log(jnp.sum(jnp.exp(zs), axis=1, keepdims=True))
  o_ref[...] = zs - lse


_tc_post = pl.pallas_call(
    _tc_post_body,
    grid=(N_NODES // ROW_BLK,),
    in_specs=[
        pl.BlockSpec((ROW_BLK, CLASSES), lambda i: (i, 0)),
        pl.BlockSpec((ROW_BLK, CLASSES), lambda i: (i, 0)),
        pl.BlockSpec((ROW_BLK, CLASSES), lambda i: (i, 0)),
    ],
    out_specs=pl.BlockSpec((ROW_BLK, CLASSES), lambda i: (i, 0)),
    out_shape=jax.ShapeDtypeStruct((N_NODES, CLASSES), jnp.float32),
)


def kernel(x, adj_t, W1_rel, W1_root, b1, W2_rel, W2_root, b2):
  src = adj_t[0].astype(jnp.int32)
  dst = adj_t[1].astype(jnp.int32)
  pad = E_PAD - N_EDGES
  packed = src | (dst << 16)
  pk3 = jnp.concatenate(
      [packed, jnp.full((pad,), N_NODES << 16, jnp.int32)]).reshape(
          NW, WPW, WIN)

  y1a, y1b, xr = _tc_pre(x, W1_rel[:W64], W1_rel[W64:], W1_root,
                         b1.reshape(1, HIDDEN))
  p1 = _sc_partials_l1(y1a, y1b, pk3)
  y2, hr = _tc_mid(p1[0, 0], p1[1, 0], p1[0, 1], p1[1, 1], xr,
                   W2_rel, W2_root, b2.reshape(1, CLASSES))
  p2 = _sc_partials_l2(y2, pk3)
  return _tc_post(p2[0, 0], p2[1, 0], hr)


# speedup vs baseline: 1.0970x; 1.0970x over previous
import functools

import jax
import jax.numpy as jnp
from jax import lax
from jax.experimental import pallas as pl
from jax.experimental.pallas import tpu as pltpu
from jax.experimental.pallas import tpu_sc as plsc

N_NODES = 10000
N_EDGES = 320000
FEATURE = 128
HIDDEN = 128
CLASSES = 64

NC = 2
NS = 16
NW = NC * NS
WIN = 125
WPW = 80
ACC_ROWS = N_NODES + 16
STRIPE = ACC_ROWS // NS

ROW_BLK = 1000


def _make_sc_partials(width):
  mesh = plsc.VectorSubcoreMesh(core_axis_name="c", subcore_axis_name="s")

  @functools.partial(
      pl.kernel,
      mesh=mesh,
      compiler_params=pltpu.CompilerParams(use_tc_tiling_on_sc=False),
      out_type=jax.ShapeDtypeStruct((NC, N_NODES, width), jnp.float32),
      scratch_types=[
          pltpu.VMEM((WPW, WIN), jnp.int32),
          pltpu.VMEM((WPW, WIN), jnp.int32),
          pltpu.VMEM((WIN, width), jnp.float32),
          pltpu.VMEM((WIN, width), jnp.float32),
          pltpu.VMEM_SHARED((N_NODES, width), jnp.float32),
          pltpu.VMEM_SHARED((ACC_ROWS, width), jnp.float32),
          pltpu.SemaphoreType.DMA,
          pltpu.SemaphoreType.DMA,
      ],
  )
  def sc_kernel(y_hbm, src_hbm, dst_hbm, out_hbm, src_v, dst_v, buf, buf_b,
                y_s, acc, sem, sem_b):
    c = lax.axis_index("c")
    s = lax.axis_index("s")
    wid = c * NS + s

    pltpu.sync_copy(src_hbm.at[wid], src_v)
    pltpu.sync_copy(dst_hbm.at[wid], dst_v)
    load = N_NODES // NS
    pltpu.sync_copy(y_hbm.at[pl.ds(s * load, load)],
                    y_s.at[pl.ds(s * load, load)])

    zero16 = jnp.zeros((16,), jnp.float32)

    @pl.loop(0, WIN)
    def _(r):
      for cc in range(width // 16):
        buf[r, pl.ds(cc * 16, 16)] = zero16

    base = s * STRIPE
    for k in range(STRIPE // WIN):
      pltpu.sync_copy(buf, acc.at[pl.ds(base + k * WIN, WIN)])
    rem = STRIPE % WIN
    if rem:
      pltpu.sync_copy(buf.at[pl.ds(0, rem)],
                      acc.at[pl.ds(base + (STRIPE // WIN) * WIN, rem)])

    plsc.subcore_barrier()

    def _fire(j, b, sm):
      pltpu.make_async_copy(y_s.at[src_v.at[j]], b, sm).start()

    def _drain(j, b, sm):
      pltpu.make_async_copy(y_s.at[src_v.at[j]], b, sm).wait()

    _fire(0, buf, sem)

    @pl.loop(0, WPW // 2)
    def _(p):
      j = 2 * p
      _fire(j + 1, buf_b, sem_b)
      _drain(j, buf, sem)
      pltpu.sync_copy(buf, acc.at[dst_v.at[j]], add=True)

      @pl.when(j + 2 < WPW)
      def _():
        _fire(j + 2, buf, sem)

      _drain(j + 1, buf_b, sem_b)
      pltpu.sync_copy(buf_b, acc.at[dst_v.at[j + 1]], add=True)

    plsc.subcore_barrier()

    @pl.when(s == 0)
    def _():
      pltpu.sync_copy(acc.at[pl.ds(0, N_NODES)], out_hbm.at[c])

  return sc_kernel


W64 = 64
_sc_partials64 = _make_sc_partials(W64)


def _dotT(a, w):
  return lax.dot_general(a, w, (((1,), (1,)), ((), ())),
                         preferred_element_type=jnp.float32)


def _tc_pre_body(x_ref, wra_ref, wrb_ref, wroot_ref, b_ref,
                 ya_ref, yb_ref, xr_ref):
  xb = x_ref[...]
  ya_ref[...] = _dotT(xb, wra_ref[...])
  yb_ref[...] = _dotT(xb, wrb_ref[...])
  xr_ref[...] = _dotT(xb, wroot_ref[...]) + b_ref[...]


_tc_pre = pl.pallas_call(
    _tc_pre_body,
    grid=(N_NODES // ROW_BLK,),
    in_specs=[
        pl.BlockSpec((ROW_BLK, FEATURE), lambda i: (i, 0)),
        pl.BlockSpec((W64, FEATURE), lambda i: (0, 0)),
        pl.BlockSpec((W64, FEATURE), lambda i: (0, 0)),
        pl.BlockSpec((HIDDEN, FEATURE), lambda i: (0, 0)),
        pl.BlockSpec((1, HIDDEN), lambda i: (0, 0)),
    ],
    out_specs=[
        pl.BlockSpec((ROW_BLK, W64), lambda i: (i, 0)),
        pl.BlockSpec((ROW_BLK, W64), lambda i: (i, 0)),
        pl.BlockSpec((ROW_BLK, HIDDEN), lambda i: (i, 0)),
    ],
    out_shape=[
        jax.ShapeDtypeStruct((N_NODES, W64), jnp.float32),
        jax.ShapeDtypeStruct((N_NODES, W64), jnp.float32),
        jax.ShapeDtypeStruct((N_NODES, HIDDEN), jnp.float32),
    ],
)


def _tc_mid_body(pa_ref, pb_ref, xr_ref,
                 wrel_ref, wroot_ref, b_ref, y2_ref, hr_ref):
  ha = pa_ref[0] + pa_ref[1] + xr_ref[:, :W64]
  hb = pb_ref[0] + pb_ref[1] + xr_ref[:, W64:]
  h = jnp.maximum(jnp.concatenate([ha, hb], axis=1), 0.0)
  y2_ref[...] = _dotT(h, wrel_ref[...])
  hr_ref[...] = _dotT(h, wroot_ref[...]) + b_ref[...]


_tc_mid = pl.pallas_call(
    _tc_mid_body,
    grid=(N_NODES // ROW_BLK,),
    in_specs=[
        pl.BlockSpec((NC, ROW_BLK, W64), lambda i: (0, i, 0)),
        pl.BlockSpec((NC, ROW_BLK, W64), lambda i: (0, i, 0)),
        pl.BlockSpec((ROW_BLK, HIDDEN), lambda i: (i, 0)),
        pl.BlockSpec((CLASSES, HIDDEN), lambda i: (0, 0)),
        pl.BlockSpec((CLASSES, HIDDEN), lambda i: (0, 0)),
        pl.BlockSpec((1, CLASSES), lambda i: (0, 0)),
    ],
    out_specs=[
        pl.BlockSpec((ROW_BLK, CLASSES), lambda i: (i, 0)),
        pl.BlockSpec((ROW_BLK, CLASSES), lambda i: (i, 0)),
    ],
    out_shape=[
        jax.ShapeDtypeStruct((N_NODES, CLASSES), jnp.float32),
        jax.ShapeDtypeStruct((N_NODES, CLASSES), jnp.float32),
    ],
)


def _tc_post_body(p_ref, hr_ref, o_ref):
  z = p_ref[0] + p_ref[1] + hr_ref[...]
  m = jnp.max(z, axis=1, keepdims=True)
  zs = z - m
  lse = jnp.log(jnp.sum(jnp.exp(zs), axis=1, keepdims=True))
  o_ref[...] = zs - lse


_tc_post = pl.pallas_call(
    _tc_post_body,
    grid=(N_NODES // ROW_BLK,),
    in_specs=[
        pl.BlockSpec((NC, ROW_BLK, CLASSES), lambda i: (0, i, 0)),
        pl.BlockSpec((ROW_BLK, CLASSES), lambda i: (i, 0)),
    ],
    out_specs=pl.BlockSpec((ROW_BLK, CLASSES), lambda i: (i, 0)),
    out_shape=jax.ShapeDtypeStruct((N_NODES, CLASSES), jnp.float32),
)


def kernel(x, adj_t, W1_rel, W1_root, b1, W2_rel, W2_root, b2):
  src3 = adj_t[0].astype(jnp.int32).reshape(NW, WPW, WIN)
  dst3 = adj_t[1].astype(jnp.int32).reshape(NW, WPW, WIN)

  y1a, y1b, xr = _tc_pre(x, W1_rel[:W64], W1_rel[W64:], W1_root,
                         b1.reshape(1, HIDDEN))
  pa = _sc_partials64(y1a, src3, dst3)
  pb = _sc_partials64(y1b, src3, dst3)
  y2, hr = _tc_mid(pa, pb, xr, W2_rel, W2_root, b2.reshape(1, CLASSES))
  p2 = _sc_partials64(y2, src3, dst3)
  return _tc_post(p2, hr)


# speedup vs baseline: 1.2824x; 1.1691x over previous
import functools

import jax
import jax.numpy as jnp
from jax import lax
from jax.experimental import pallas as pl
from jax.experimental.pallas import tpu as pltpu
from jax.experimental.pallas import tpu_sc as plsc

N_NODES = 10000
N_EDGES = 320000
FEATURE = 128
HIDDEN = 128
CLASSES = 64

NC = 2
NS = 16
NW = NC * NS
WIN = 125
WPW = 80
ACC_ROWS = N_NODES + 16
STRIPE = ACC_ROWS // NS

ROW_BLK = 1000


W64 = 64


def _make_sc_partials(n_pass):
  mesh = plsc.VectorSubcoreMesh(core_axis_name="c", subcore_axis_name="s")

  @functools.partial(
      pl.kernel,
      mesh=mesh,
      compiler_params=pltpu.CompilerParams(use_tc_tiling_on_sc=False),
      out_type=jax.ShapeDtypeStruct((NC, N_NODES, 2 * W64), jnp.float32),
      scratch_types=[
          pltpu.VMEM((WPW, WIN), jnp.int32),
          pltpu.VMEM((WPW, WIN), jnp.int32),
          pltpu.VMEM((WIN, W64), jnp.float32),
          pltpu.VMEM((WIN, W64), jnp.float32),
          pltpu.VMEM_SHARED((N_NODES, W64), jnp.float32),
          pltpu.VMEM_SHARED((ACC_ROWS, W64), jnp.float32),
          pltpu.SemaphoreType.DMA,
          pltpu.SemaphoreType.DMA,
      ],
  )
  def sc_kernel(y_hbm, edges_hbm, out_hbm, src_v, dst_v, buf, buf_b,
                y_s, acc, sem, sem_b):
    c = lax.axis_index("c")
    s = lax.axis_index("s")
    wid = c * NS + s

    pltpu.sync_copy(edges_hbm.at[0, wid], src_v)
    pltpu.sync_copy(edges_hbm.at[1, wid], dst_v)

    zero16 = jnp.zeros((16,), jnp.float32)
    load = N_NODES // NS
    base = s * STRIPE

    def _pass(t):
      pltpu.sync_copy(y_hbm.at[pl.ds(s * load, load), pl.ds(W64 * t, W64)],
                      y_s.at[pl.ds(s * load, load)])

      @pl.loop(0, WIN)
      def _(r):
        for cc in range(W64 // 16):
          buf[r, pl.ds(cc * 16, 16)] = zero16

      for k in range(STRIPE // WIN):
        pltpu.sync_copy(buf, acc.at[pl.ds(base + k * WIN, WIN)])
      rem = STRIPE % WIN
      if rem:
        pltpu.sync_copy(buf.at[pl.ds(0, rem)],
                        acc.at[pl.ds(base + (STRIPE // WIN) * WIN, rem)])

      plsc.subcore_barrier()

      def _fire(j, b, sm):
        pltpu.make_async_copy(y_s.at[src_v.at[j]], b, sm).start()

      def _drain(j, b, sm):
        pltpu.make_async_copy(y_s.at[src_v.at[j]], b, sm).wait()

      _fire(0, buf, sem)

      @pl.loop(0, WPW // 2)
      def _(p):
        j = 2 * p
        _fire(j + 1, buf_b, sem_b)
        _drain(j, buf, sem)
        pltpu.sync_copy(buf, acc.at[dst_v.at[j]], add=True)

        @pl.when(j + 2 < WPW)
        def _():
          _fire(j + 2, buf, sem)

        _drain(j + 1, buf_b, sem_b)
        pltpu.sync_copy(buf_b, acc.at[dst_v.at[j + 1]], add=True)

      plsc.subcore_barrier()

      @pl.when(s == 0)
      def _():
        pltpu.sync_copy(
            acc.at[pl.ds(0, N_NODES)],
            out_hbm.at[c, pl.ds(0, N_NODES), pl.ds(W64 * t, W64)])

      plsc.subcore_barrier()

    for t in range(n_pass):
      _pass(t)

  return sc_kernel


_sc_partials_l1 = _make_sc_partials(2)
_sc_partials_l2 = _make_sc_partials(1)


def _dotT(a, w):
  return lax.dot_general(a, w, (((1,), (1,)), ((), ())),
                         preferred_element_type=jnp.float32)


def _tc_pre_body(x_ref, wrel_ref, wroot_ref, b_ref, y_ref, xr_ref):
  xb = x_ref[...]
  y_ref[...] = _dotT(xb, wrel_ref[...])
  xr_ref[...] = _dotT(xb, wroot_ref[...]) + b_ref[...]


_tc_pre = pl.pallas_call(
    _tc_pre_body,
    grid=(N_NODES // ROW_BLK,),
    in_specs=[
        pl.BlockSpec((ROW_BLK, FEATURE), lambda i: (i, 0)),
        pl.BlockSpec((HIDDEN, FEATURE), lambda i: (0, 0)),
        pl.BlockSpec((HIDDEN, FEATURE), lambda i: (0, 0)),
        pl.BlockSpec((1, HIDDEN), lambda i: (0, 0)),
    ],
    out_specs=[
        pl.BlockSpec((ROW_BLK, HIDDEN), lambda i: (i, 0)),
        pl.BlockSpec((ROW_BLK, HIDDEN), lambda i: (i, 0)),
    ],
    out_shape=[
        jax.ShapeDtypeStruct((N_NODES, HIDDEN), jnp.float32),
        jax.ShapeDtypeStruct((N_NODES, HIDDEN), jnp.float32),
    ],
)


def _tc_mid_body(p_ref, xr_ref, wrel_ref, wroot_ref, b_ref,
                 y2_ref, hr_ref):
  h = jnp.maximum(p_ref[0] + p_ref[1] + xr_ref[...], 0.0)
  y2 = _dotT(h, wrel_ref[...])
  y2_ref[...] = jnp.concatenate(
      [y2, jnp.zeros((ROW_BLK, 2 * W64 - CLASSES), jnp.float32)], axis=1)
  hr_ref[...] = _dotT(h, wroot_ref[...]) + b_ref[...]


_tc_mid = pl.pallas_call(
    _tc_mid_body,
    grid=(N_NODES // ROW_BLK,),
    in_specs=[
        pl.BlockSpec((NC, ROW_BLK, 2 * W64), lambda i: (0, i, 0)),
        pl.BlockSpec((ROW_BLK, HIDDEN), lambda i: (i, 0)),
        pl.BlockSpec((CLASSES, HIDDEN), lambda i: (0, 0)),
        pl.BlockSpec((CLASSES, HIDDEN), lambda i: (0, 0)),
        pl.BlockSpec((1, CLASSES), lambda i: (0, 0)),
    ],
    out_specs=[
        pl.BlockSpec((ROW_BLK, 2 * W64), lambda i: (i, 0)),
        pl.BlockSpec((ROW_BLK, CLASSES), lambda i: (i, 0)),
    ],
    out_shape=[
        jax.ShapeDtypeStruct((N_NODES, 2 * W64), jnp.float32),
        jax.ShapeDtypeStruct((N_NODES, CLASSES), jnp.float32),
    ],
)


def _tc_post_body(p_ref, hr_ref, o_ref):
  z = p_ref[0, :, :CLASSES] + p_ref[1, :, :CLASSES] + hr_ref[...]
  m = jnp.max(z, axis=1, keepdims=True)
  zs = z - m
  lse = jnp.log(jnp.sum(jnp.exp(zs), axis=1, keepdims=True))
  o_ref[...] = zs - lse


_tc_post = pl.pallas_call(
    _tc_post_body,
    grid=(N_NODES // ROW_BLK,),
    in_specs=[
        pl.BlockSpec((NC, ROW_BLK, 2 * W64), lambda i: (0, i, 0)),
        pl.BlockSpec((ROW_BLK, CLASSES), lambda i: (i, 0)),
    ],
    out_specs=pl.BlockSpec((ROW_BLK, CLASSES), lambda i: (i, 0)),
    out_shape=jax.ShapeDtypeStruct((N_NODES, CLASSES), jnp.float32),
)


def kernel(x, adj_t, W1_rel, W1_root, b1, W2_rel, W2_root, b2):
  edges = adj_t.astype(jnp.int32).reshape(2, NW, WPW, WIN)

  y1, xr = _tc_pre(x, W1_rel, W1_root, b1.reshape(1, HIDDEN))
  p1 = _sc_partials_l1(y1, edges)
  y2, hr = _tc_mid(p1, xr, W2_rel, W2_root, b2.reshape(1, CLASSES))
  p2 = _sc_partials_l2(y2, edges)
  return _tc_post(p2, hr)


# speedup vs baseline: 1.2956x; 1.0102x over previous
import functools

import jax
import jax.numpy as jnp
from jax import lax
from jax.experimental import pallas as pl
from jax.experimental.pallas import tpu as pltpu
from jax.experimental.pallas import tpu_sc as plsc

N_NODES = 10000
N_EDGES = 320000
FEATURE = 128
HIDDEN = 128
CLASSES = 64

NC = 2
NS = 16
NW = NC * NS
WIN = 128
NWIN = N_EDGES // WIN
WPC = NWIN // NC
WPW = 78
NEXTRA = WPC - NS * WPW
ACC_ROWS = N_NODES + 16
STRIPE = ACC_ROWS // NS

ROW_BLK = 1000


W64 = 64


def _make_sc_partials(n_pass):
  mesh = plsc.VectorSubcoreMesh(core_axis_name="c", subcore_axis_name="s")

  @functools.partial(
      pl.kernel,
      mesh=mesh,
      compiler_params=pltpu.CompilerParams(use_tc_tiling_on_sc=False),
      out_type=jax.ShapeDtypeStruct((NC, N_NODES, 2 * W64), jnp.float32),
      scratch_types=[
          pltpu.VMEM((WPW + 1, WIN), jnp.int32),
          pltpu.VMEM((WPW + 1, WIN), jnp.int32),
          pltpu.VMEM((WIN, W64), jnp.float32),
          pltpu.VMEM((WIN, W64), jnp.float32),
          pltpu.VMEM_SHARED((N_NODES, W64), jnp.float32),
          pltpu.VMEM_SHARED((ACC_ROWS, W64), jnp.float32),
          pltpu.SemaphoreType.DMA,
          pltpu.SemaphoreType.DMA,
      ],
  )
  def sc_kernel(y_hbm, edges_hbm, out_hbm, src_v, dst_v, buf, buf_b,
                y_s, acc, sem, sem_b):
    c = lax.axis_index("c")
    s = lax.axis_index("s")

    wbase = c * WPC + s * WPW
    has_extra = s < NEXTRA
    pltpu.sync_copy(edges_hbm.at[0, pl.ds(wbase, WPW)],
                    src_v.at[pl.ds(0, WPW)])
    pltpu.sync_copy(edges_hbm.at[1, pl.ds(wbase, WPW)],
                    dst_v.at[pl.ds(0, WPW)])

    @pl.when(has_extra)
    def _():
      xw = c * WPC + NS * WPW + s
      pltpu.sync_copy(edges_hbm.at[0, pl.ds(xw, 1)], src_v.at[pl.ds(WPW, 1)])
      pltpu.sync_copy(edges_hbm.at[1, pl.ds(xw, 1)], dst_v.at[pl.ds(WPW, 1)])

    zero16 = jnp.zeros((16,), jnp.float32)
    load = N_NODES // NS
    base = s * STRIPE

    def _pass(t):
      pltpu.sync_copy(y_hbm.at[pl.ds(s * load, load), pl.ds(W64 * t, W64)],
                      y_s.at[pl.ds(s * load, load)])

      @pl.loop(0, WIN)
      def _(r):
        for cc in range(W64 // 16):
          buf[r, pl.ds(cc * 16, 16)] = zero16

      for k in range(STRIPE // WIN):
        pltpu.sync_copy(buf, acc.at[pl.ds(base + k * WIN, WIN)])
      rem = STRIPE % WIN
      if rem:
        pltpu.sync_copy(buf.at[pl.ds(0, rem)],
                        acc.at[pl.ds(base + (STRIPE // WIN) * WIN, rem)])

      plsc.subcore_barrier()

      def _fire(j, b, sm):
        pltpu.make_async_copy(y_s.at[src_v.at[j]], b, sm).start()

      def _drain(j, b, sm):
        pltpu.make_async_copy(y_s.at[src_v.at[j]], b, sm).wait()

      _fire(0, buf, sem)

      @pl.loop(0, WPW // 2)
      def _(p):
        j = 2 * p
        _fire(j + 1, buf_b, sem_b)
        _drain(j, buf, sem)
        pltpu.sync_copy(buf, acc.at[dst_v.at[j]], add=True)

        @pl.when(j + 2 < WPW)
        def _():
          _fire(j + 2, buf, sem)

        _drain(j + 1, buf_b, sem_b)
        pltpu.sync_copy(buf_b, acc.at[dst_v.at[j + 1]], add=True)

      @pl.when(has_extra)
      def _():
        _fire(WPW, buf, sem)
        _drain(WPW, buf, sem)
        pltpu.sync_copy(buf, acc.at[dst_v.at[WPW]], add=True)

      plsc.subcore_barrier()

      @pl.when(s == 0)
      def _():
        pltpu.sync_copy(
            acc.at[pl.ds(0, N_NODES)],
            out_hbm.at[c, pl.ds(0, N_NODES), pl.ds(W64 * t, W64)])

      plsc.subcore_barrier()

    for t in range(n_pass):
      _pass(t)

  return sc_kernel


_sc_partials_l1 = _make_sc_partials(2)
_sc_partials_l2 = _make_sc_partials(1)


def _dotT(a, w):
  return lax.dot_general(a, w, (((1,), (1,)), ((), ())),
                         preferred_element_type=jnp.float32)


def _tc_pre_body(x_ref, wrel_ref, wroot_ref, b_ref, y_ref, xr_ref):
  xb = x_ref[...]
  y_ref[...] = _dotT(xb, wrel_ref[...])
  xr_ref[...] = _dotT(xb, wroot_ref[...]) + b_ref[...]


_tc_pre = pl.pallas_call(
    _tc_pre_body,
    grid=(N_NODES // ROW_BLK,),
    in_specs=[
        pl.BlockSpec((ROW_BLK, FEATURE), lambda i: (i, 0)),
        pl.BlockSpec((HIDDEN, FEATURE), lambda i: (0, 0)),
        pl.BlockSpec((HIDDEN, FEATURE), lambda i: (0, 0)),
        pl.BlockSpec((1, HIDDEN), lambda i: (0, 0)),
    ],
    out_specs=[
        pl.BlockSpec((ROW_BLK, HIDDEN), lambda i: (i, 0)),
        pl.BlockSpec((ROW_BLK, HIDDEN), lambda i: (i, 0)),
    ],
    out_shape=[
        jax.ShapeDtypeStruct((N_NODES, HIDDEN), jnp.float32),
        jax.ShapeDtypeStruct((N_NODES, HIDDEN), jnp.float32),
    ],
)


def _tc_mid_body(p_ref, xr_ref, wrel_ref, wroot_ref, b_ref,
                 y2_ref, hr_ref):
  h = jnp.maximum(p_ref[0] + p_ref[1] + xr_ref[...], 0.0)
  y2 = _dotT(h, wrel_ref[...])
  y2_ref[...] = jnp.concatenate(
      [y2, jnp.zeros((ROW_BLK, 2 * W64 - CLASSES), jnp.float32)], axis=1)
  hr_ref[...] = _dotT(h, wroot_ref[...]) + b_ref[...]


_tc_mid = pl.pallas_call(
    _tc_mid_body,
    grid=(N_NODES // ROW_BLK,),
    in_specs=[
        pl.BlockSpec((NC, ROW_BLK, 2 * W64), lambda i: (0, i, 0)),
        pl.BlockSpec((ROW_BLK, HIDDEN), lambda i: (i, 0)),
        pl.BlockSpec((CLASSES, HIDDEN), lambda i: (0, 0)),
        pl.BlockSpec((CLASSES, HIDDEN), lambda i: (0, 0)),
        pl.BlockSpec((1, CLASSES), lambda i: (0, 0)),
    ],
    out_specs=[
        pl.BlockSpec((ROW_BLK, 2 * W64), lambda i: (i, 0)),
        pl.BlockSpec((ROW_BLK, CLASSES), lambda i: (i, 0)),
    ],
    out_shape=[
        jax.ShapeDtypeStruct((N_NODES, 2 * W64), jnp.float32),
        jax.ShapeDtypeStruct((N_NODES, CLASSES), jnp.float32),
    ],
)


def _tc_post_body(p_ref, hr_ref, o_ref):
  z = p_ref[0, :, :CLASSES] + p_ref[1, :, :CLASSES] + hr_ref[...]
  m = jnp.max(z, axis=1, keepdims=True)
  zs = z - m
  lse = jnp.log(jnp.sum(jnp.exp(zs), axis=1, keepdims=True))
  o_ref[...] = zs - lse


_tc_post = pl.pallas_call(
    _tc_post_body,
    grid=(N_NODES // ROW_BLK,),
    in_specs=[
        pl.BlockSpec((NC, ROW_BLK, 2 * W64), lambda i: (0, i, 0)),
        pl.BlockSpec((ROW_BLK, CLASSES), lambda i: (i, 0)),
    ],
    out_specs=pl.BlockSpec((ROW_BLK, CLASSES), lambda i: (i, 0)),
    out_shape=jax.ShapeDtypeStruct((N_NODES, CLASSES), jnp.float32),
)


def kernel(x, adj_t, W1_rel, W1_root, b1, W2_rel, W2_root, b2):
  edges = adj_t.astype(jnp.int32).reshape(2, NWIN, WIN)

  y1, xr = _tc_pre(x, W1_rel, W1_root, b1.reshape(1, HIDDEN))
  p1 = _sc_partials_l1(y1, edges)
  y2, hr = _tc_mid(p1, xr, W2_rel, W2_root, b2.reshape(1, CLASSES))
  p2 = _sc_partials_l2(y2, edges)
  return _tc_post(p2, hr)


# speedup vs baseline: 1.3009x; 1.0041x over previous
import functools

import jax
import jax.numpy as jnp
from jax import lax
from jax.experimental import pallas as pl
from jax.experimental.pallas import tpu as pltpu
from jax.experimental.pallas import tpu_sc as plsc

N_NODES = 10000
N_EDGES = 320000
FEATURE = 128
HIDDEN = 128
CLASSES = 64

NC = 2
NS = 16
NW = NC * NS
WIN = 128
NWIN = N_EDGES // WIN
WPC = NWIN // NC
WPW = 78
NEXTRA = WPC - NS * WPW
ACC_ROWS = N_NODES + 16
STRIPE = ACC_ROWS // NS

ROW_BLK = 1000


W64 = 64


def _make_sc_partials(n_pass):
  mesh = plsc.VectorSubcoreMesh(core_axis_name="c", subcore_axis_name="s")

  @functools.partial(
      pl.kernel,
      mesh=mesh,
      compiler_params=pltpu.CompilerParams(use_tc_tiling_on_sc=False),
      out_type=jax.ShapeDtypeStruct((NC, N_NODES, 2 * W64), jnp.float32),
      scratch_types=[
          pltpu.VMEM((WPW + 1, WIN), jnp.int32),
          pltpu.VMEM((WPW + 1, WIN), jnp.int32),
          pltpu.VMEM((WIN, W64), jnp.float32),
          pltpu.VMEM((WIN, W64), jnp.float32),
          pltpu.VMEM_SHARED((N_NODES, W64), jnp.float32),
          pltpu.VMEM_SHARED((ACC_ROWS, W64), jnp.float32),
          pltpu.SemaphoreType.DMA,
          pltpu.SemaphoreType.DMA,
      ],
  )
  def sc_kernel(y_hbm, edges_hbm, out_hbm, src_v, dst_v, buf, buf_b,
                y_s, acc, sem, sem_b):
    c = lax.axis_index("c")
    s = lax.axis_index("s")

    wbase = c * WPC + s * WPW
    has_extra = s < NEXTRA
    pltpu.sync_copy(edges_hbm.at[0, pl.ds(wbase, WPW)],
                    src_v.at[pl.ds(0, WPW)])
    pltpu.sync_copy(edges_hbm.at[1, pl.ds(wbase, WPW)],
                    dst_v.at[pl.ds(0, WPW)])

    @pl.when(has_extra)
    def _():
      xw = c * WPC + NS * WPW + s
      pltpu.sync_copy(edges_hbm.at[0, pl.ds(xw, 1)], src_v.at[pl.ds(WPW, 1)])
      pltpu.sync_copy(edges_hbm.at[1, pl.ds(xw, 1)], dst_v.at[pl.ds(WPW, 1)])

    zero16 = jnp.zeros((16,), jnp.float32)
    load = N_NODES // NS
    base = s * STRIPE

    def _pass(t):
      pltpu.sync_copy(y_hbm.at[pl.ds(s * load, load), pl.ds(W64 * t, W64)],
                      y_s.at[pl.ds(s * load, load)])

      @pl.loop(0, WIN)
      def _(r):
        for cc in range(W64 // 16):
          buf[r, pl.ds(cc * 16, 16)] = zero16

      for k in range(STRIPE // WIN):
        pltpu.sync_copy(buf, acc.at[pl.ds(base + k * WIN, WIN)])
      rem = STRIPE % WIN
      if rem:
        pltpu.sync_copy(buf.at[pl.ds(0, rem)],
                        acc.at[pl.ds(base + (STRIPE // WIN) * WIN, rem)])

      plsc.subcore_barrier()

      def _fire(j, b, sm):
        pltpu.make_async_copy(y_s.at[src_v.at[j]], b, sm).start()

      def _drain(j, b, sm):
        pltpu.make_async_copy(y_s.at[src_v.at[j]], b, sm).wait()

      _fire(0, buf, sem)

      @pl.loop(0, WPW // 2)
      def _(p):
        j = 2 * p
        _fire(j + 1, buf_b, sem_b)
        _drain(j, buf, sem)
        pltpu.sync_copy(buf, acc.at[dst_v.at[j]], add=True)

        @pl.when(j + 2 < WPW)
        def _():
          _fire(j + 2, buf, sem)

        _drain(j + 1, buf_b, sem_b)
        pltpu.sync_copy(buf_b, acc.at[dst_v.at[j + 1]], add=True)

      @pl.when(has_extra)
      def _():
        _fire(WPW, buf, sem)
        _drain(WPW, buf, sem)
        pltpu.sync_copy(buf, acc.at[dst_v.at[WPW]], add=True)

      plsc.subcore_barrier()

      @pl.when(s == 0)
      def _():
        pltpu.sync_copy(
            acc.at[pl.ds(0, N_NODES)],
            out_hbm.at[c, pl.ds(0, N_NODES), pl.ds(W64 * t, W64)])

      plsc.subcore_barrier()

    for t in range(n_pass):
      _pass(t)

  return sc_kernel


_sc_partials_l1 = _make_sc_partials(2)
_sc_partials_l2 = _make_sc_partials(1)


def _dotT(a, w):
  return lax.dot_general(a, w, (((1,), (1,)), ((), ())),
                         preferred_element_type=jnp.float32)


def _tc_y1_body(x_ref, wrel_ref, y_ref):
  y_ref[...] = _dotT(x_ref[...], wrel_ref[...])


_tc_y1 = pl.pallas_call(
    _tc_y1_body,
    grid=(N_NODES // ROW_BLK,),
    in_specs=[
        pl.BlockSpec((ROW_BLK, FEATURE), lambda i: (i, 0)),
        pl.BlockSpec((HIDDEN, FEATURE), lambda i: (0, 0)),
    ],
    out_specs=pl.BlockSpec((ROW_BLK, HIDDEN), lambda i: (i, 0)),
    out_shape=jax.ShapeDtypeStruct((N_NODES, HIDDEN), jnp.float32),
)


def _tc_xr_body(x_ref, wroot_ref, b_ref, xr_ref):
  xr_ref[...] = _dotT(x_ref[...], wroot_ref[...]) + b_ref[...]


_tc_xr = pl.pallas_call(
    _tc_xr_body,
    grid=(N_NODES // ROW_BLK,),
    in_specs=[
        pl.BlockSpec((ROW_BLK, FEATURE), lambda i: (i, 0)),
        pl.BlockSpec((HIDDEN, FEATURE), lambda i: (0, 0)),
        pl.BlockSpec((1, HIDDEN), lambda i: (0, 0)),
    ],
    out_specs=pl.BlockSpec((ROW_BLK, HIDDEN), lambda i: (i, 0)),
    out_shape=jax.ShapeDtypeStruct((N_NODES, HIDDEN), jnp.float32),
)


def _tc_mid_body(p_ref, xr_ref, wrel_ref, wroot_ref, b_ref,
                 y2_ref, hr_ref):
  h = jnp.maximum(p_ref[0] + p_ref[1] + xr_ref[...], 0.0)
  y2 = _dotT(h, wrel_ref[...])
  y2_ref[...] = jnp.concatenate(
      [y2, jnp.zeros((ROW_BLK, 2 * W64 - CLASSES), jnp.float32)], axis=1)
  hr_ref[...] = _dotT(h, wroot_ref[...]) + b_ref[...]


_tc_mid = pl.pallas_call(
    _tc_mid_body,
    grid=(N_NODES // ROW_BLK,),
    in_specs=[
        pl.BlockSpec((NC, ROW_BLK, 2 * W64), lambda i: (0, i, 0)),
        pl.BlockSpec((ROW_BLK, HIDDEN), lambda i: (i, 0)),
        pl.BlockSpec((CLASSES, HIDDEN), lambda i: (0, 0)),
        pl.BlockSpec((CLASSES, HIDDEN), lambda i: (0, 0)),
        pl.BlockSpec((1, CLASSES), lambda i: (0, 0)),
    ],
    out_specs=[
        pl.BlockSpec((ROW_BLK, 2 * W64), lambda i: (i, 0)),
        pl.BlockSpec((ROW_BLK, CLASSES), lambda i: (i, 0)),
    ],
    out_shape=[
        jax.ShapeDtypeStruct((N_NODES, 2 * W64), jnp.float32),
        jax.ShapeDtypeStruct((N_NODES, CLASSES), jnp.float32),
    ],
)


def _tc_post_body(p_ref, hr_ref, o_ref):
  z = p_ref[0, :, :CLASSES] + p_ref[1, :, :CLASSES] + hr_ref[...]
  m = jnp.max(z, axis=1, keepdims=True)
  zs = z - m
  lse = jnp.log(jnp.sum(jnp.exp(zs), axis=1, keepdims=True))
  o_ref[...] = zs - lse


_tc_post = pl.pallas_call(
    _tc_post_body,
    grid=(N_NODES // ROW_BLK,),
    in_specs=[
        pl.BlockSpec((NC, ROW_BLK, 2 * W64), lambda i: (0, i, 0)),
        pl.BlockSpec((ROW_BLK, CLASSES), lambda i: (i, 0)),
    ],
    out_specs=pl.BlockSpec((ROW_BLK, CLASSES), lambda i: (i, 0)),
    out_shape=jax.ShapeDtypeStruct((N_NODES, CLASSES), jnp.float32),
)


def kernel(x, adj_t, W1_rel, W1_root, b1, W2_rel, W2_root, b2):
  edges = adj_t.astype(jnp.int32).reshape(2, NWIN, WIN)

  y1 = _tc_y1(x, W1_rel)
  p1 = _sc_partials_l1(y1, edges)
  xr = _tc_xr(x, W1_root, b1.reshape(1, HIDDEN))
  y2, hr = _tc_mid(p1, xr, W2_rel, W2_root, b2.reshape(1, CLASSES))
  p2 = _sc_partials_l2(y2, edges)
  return _tc_post(p2, hr)
